# baseline (speedup 1.0000x reference)
import functools

import jax
import jax.numpy as jnp
from jax import lax
from jax.experimental import pallas as pl
from jax.experimental.pallas import tpu as pltpu
from jax.experimental.pallas import tpu_sc as plsc

ROWS = 128
N = 32768
NBINS = 2048
K = 8
CHUNK = N // K
CHUNK_VREGS = CHUNK // 16
SIGN = -2147483648


def _keys(v_f32):
    b = plsc.bitcast(v_f32, jnp.int32)
    m = lax.shift_right_arithmetic(b, 31)
    return lax.bitwise_xor(b, lax.bitwise_or(m, jnp.int32(SIGN)))


def _chunk_prefixes(hs):
    t01 = hs[0] + hs[1]
    t23 = hs[2] + hs[3]
    t45 = hs[4] + hs[5]
    t67 = hs[6] + hs[7]
    pre4 = t01 + t23
    t4567 = t45 + t67
    pre = [
        None,
        hs[0],
        t01,
        t01 + hs[2],
        pre4,
        pre4 + hs[4],
        pre4 + t45,
        pre4 + t45 + hs[6],
    ]
    return pre, pre4 + t4567


def _scan_step(load_h, store_off, carry, zero_src):
    hs = [load_h(c) for c in range(K)]
    pre, total = _chunk_prefixes(hs)
    incl = plsc.cumsum(total)
    base = incl - total + carry
    store_off(0, base)
    for c in range(1, K):
        store_off(c, base + pre[c])
    if zero_src is not None:
        zeros = jnp.zeros((16,), jnp.int32)
        for c in range(K):
            zero_src(c, zeros)
    return carry + jnp.sum(total)


def _scan_ga_inplace(ga):
    def body(i, carry):
        for j in (4 * i, 4 * i + 1, 4 * i + 2, 4 * i + 3):
            sl = pl.ds(j * 16, 16)
            carry = _scan_step(
                lambda c: ga[c][sl],
                lambda c, v: ga[c].__setitem__(sl, v),
                carry,
                None,
            )
        return carry

    lax.fori_loop(0, NBINS // 64, body, jnp.int32(0))


def _transfer_scan(gb, ga):
    def body(i, carry):
        for j in (4 * i, 4 * i + 1, 4 * i + 2, 4 * i + 3):
            sl = pl.ds(j * 16, 16)

            def gbsl(c, j=j):
                return pl.ds(c * NBINS + j * 16, 16)

            carry = _scan_step(
                lambda c: gb[gbsl(c)],
                lambda c, v: ga[c].__setitem__(sl, v),
                carry,
                lambda c, z: gb.__setitem__(gbsl(c), z),
            )
        return carry

    lax.fori_loop(0, NBINS // 64, body, jnp.int32(0))


def _perm_sweep(src_digit_payload, dst, ga, gb, next_digit, cast_f32):
    ones = jnp.full((16,), 1, jnp.int32)

    def load_dp(i):
        dps = [src_digit_payload(c, i) for c in range(K)]
        return tuple(d for d, _p in dps) + tuple(p for _d, p in dps)

    def process(carry):
        ds, ps = carry[:K], carry[K:]
        scans = [plsc.scan_count(d) for d in ds]
        bases = [plsc.load_gather(ga[c], [ds[c]]) for c in range(K)]
        poss = [bases[c] + scans[c][0] - 1 for c in range(K)]
        for c in range(K):
            cnt, last = scans[c]
            val = plsc.bitcast(ps[c], jnp.float32) if cast_f32 else ps[c]
            plsc.store_scatter(dst, [poss[c]], val)
            plsc.addupdate_scatter(ga[c], [ds[c]], cnt, mask=last)
        if next_digit is not None:
            for c in range(K):
                nh = lax.bitwise_or(
                    lax.shift_left(lax.shift_right_logical(poss[c], 12), 11),
                    next_digit(ps[c]),
                )
                plsc.addupdate_scatter(gb, [nh], ones)

    def body(i, carry):
        nxt = load_dp(i + 1)
        process(carry)
        return nxt

    last_carry = lax.fori_loop(0, CHUNK_VREGS - 1, body, load_dp(0))
    process(last_carry)


def _d2(p):
    return lax.bitwise_and(p, jnp.int32(0x7FF))


def _d3(p):
    return lax.bitwise_and(p, jnp.int32(0x3FF))


def _body(x_hbm, out_hbm, xb, b, gb, *ga):
    ga = list(ga)
    cid = lax.axis_index("c")
    sid = lax.axis_index("s")
    wid = sid * 2 + cid

    zeros = jnp.zeros((16,), jnp.int32)

    @plsc.parallel_loop(0, K * NBINS // 16, unroll=8)
    def _zb_loop(i):
        gb[pl.ds(i * 16, 16)] = zeros

    def do_row(rr, _):
        row = wid * 4 + rr
        pltpu.sync_copy(x_hbm.at[row], xb)

        @plsc.parallel_loop(0, NBINS // 16, unroll=4)
        def _za_loop(i):
            sl = pl.ds(i * 16, 16)
            for c in range(K):
                ga[c][sl] = zeros

        def src1(c, i):
            key = _keys(xb[pl.ds(c * CHUNK + i * 16, 16)])
            d = lax.bitwise_and(key, jnp.int32(0x7FF))
            p = lax.shift_right_logical(key, 11)
            if c < K // 2:
                p = lax.bitwise_or(p, jnp.int32(SIGN))
            return d, p

        def src2(c, i):
            w = b[pl.ds(c * CHUNK + i * 16, 16)]
            d = lax.bitwise_and(w, jnp.int32(0x7FF))
            p = lax.shift_right_logical(w, 11)
            return d, p

        def src3(c, i):
            w = plsc.bitcast(xb[pl.ds(c * CHUNK + i * 16, 16)], jnp.int32)
            d = lax.bitwise_and(w, jnp.int32(0x3FF))
            return d, lax.shift_right_logical(w, 20)

        ones = jnp.full((16,), 1, jnp.int32)

        @plsc.parallel_loop(0, CHUNK_VREGS, unroll=4)
        def _h1_loop(i):
            ds = [src1(c, i)[0] for c in range(K)]
            for c in range(K):
                plsc.addupdate_scatter(ga[c], [ds[c]], ones)
        _scan_ga_inplace(ga)
        _perm_sweep(src1, b, ga, gb, _d2, cast_f32=False)

        _transfer_scan(gb, ga)
        _perm_sweep(src2, xb, ga, gb, _d3, cast_f32=True)

        _transfer_scan(gb, ga)
        _perm_sweep(src3, b, ga, None, None, cast_f32=False)

        pltpu.sync_copy(b, out_hbm.at[row])
        return 0

    lax.fori_loop(0, ROWS // 32, do_row, 0)


@jax.jit
def _feature_select(x):
    mesh = plsc.VectorSubcoreMesh(core_axis_name="c", subcore_axis_name="s")
    run = functools.partial(
        pl.kernel,
        out_type=jax.ShapeDtypeStruct((ROWS, N), jnp.int32),
        mesh=mesh,
        scratch_types=[
            pltpu.VMEM((N,), jnp.float32),
            pltpu.VMEM((N,), jnp.int32),
            pltpu.VMEM((K * NBINS,), jnp.int32),
        ]
        + [pltpu.VMEM((NBINS,), jnp.int32) for _ in range(K)],
        compiler_params=pltpu.CompilerParams(needs_layout_passes=False),
    )(_body)
    return run(x)


def kernel(x):
    return _feature_select(x)


# speedup vs baseline: 1.0233x; 1.0233x over previous
import functools

import jax
import jax.numpy as jnp
from jax import lax
from jax.experimental import pallas as pl
from jax.experimental.pallas import tpu as pltpu
from jax.experimental.pallas import tpu_sc as plsc

ROWS = 128
N = 32768
NBINS = 2048
K = 8
CHUNK = N // K
CHUNK_VREGS = CHUNK // 16
SIGN = -2147483648


def _keys(v_f32):
    b = plsc.bitcast(v_f32, jnp.int32)
    m = lax.shift_right_arithmetic(b, 31)
    return lax.bitwise_xor(b, lax.bitwise_or(m, jnp.int32(SIGN)))


def _chunk_prefixes(hs):
    t01 = hs[0] + hs[1]
    t23 = hs[2] + hs[3]
    t45 = hs[4] + hs[5]
    t67 = hs[6] + hs[7]
    pre4 = t01 + t23
    t4567 = t45 + t67
    pre = [
        None,
        hs[0],
        t01,
        t01 + hs[2],
        pre4,
        pre4 + hs[4],
        pre4 + t45,
        pre4 + t45 + hs[6],
    ]
    return pre, pre4 + t4567


def _scan_step(load_h, store_off, carry, zero_src):
    hs = [load_h(c) for c in range(K)]
    pre, total = _chunk_prefixes(hs)
    incl = plsc.cumsum(total)
    base = incl - total + carry
    store_off(0, base)
    for c in range(1, K):
        store_off(c, base + pre[c])
    if zero_src is not None:
        zeros = jnp.zeros((16,), jnp.int32)
        for c in range(K):
            zero_src(c, zeros)
    return carry + jnp.sum(total)


def _scan_ga_inplace(ga):
    def body(i, carry):
        for j in (4 * i, 4 * i + 1, 4 * i + 2, 4 * i + 3):
            sl = pl.ds(j * 16, 16)
            carry = _scan_step(
                lambda c: ga[c][sl],
                lambda c, v: ga[c].__setitem__(sl, v),
                carry,
                None,
            )
        return carry

    lax.fori_loop(0, NBINS // 64, body, jnp.int32(0))


def _transfer_scan(gb, ga):
    def body(i, carry):
        for j in (4 * i, 4 * i + 1, 4 * i + 2, 4 * i + 3):
            sl = pl.ds(j * 16, 16)

            def gbsl(c, j=j):
                return pl.ds(c * NBINS + j * 16, 16)

            carry = _scan_step(
                lambda c: gb[gbsl(c)],
                lambda c, v: ga[c].__setitem__(sl, v),
                carry,
                lambda c, z: gb.__setitem__(gbsl(c), z),
            )
        return carry

    lax.fori_loop(0, NBINS // 64, body, jnp.int32(0))


def _perm_sweep(src_digit_payload, dst, ga, gb, next_digit, cast_f32):
    ones = jnp.full((16,), 1, jnp.int32)

    def load_dp(i):
        dps = [src_digit_payload(c, i) for c in range(K)]
        return tuple(d for d, _p in dps) + tuple(p for _d, p in dps)

    def process(carry):
        ds, ps = carry[:K], carry[K:]
        scans = [plsc.scan_count(d) for d in ds]
        bases = [plsc.load_gather(ga[c], [ds[c]]) for c in range(K)]
        poss = [bases[c] + scans[c][0] - 1 for c in range(K)]
        for c in range(K):
            cnt, last = scans[c]
            val = plsc.bitcast(ps[c], jnp.float32) if cast_f32 else ps[c]
            plsc.store_scatter(dst, [poss[c]], val)
            plsc.addupdate_scatter(ga[c], [ds[c]], cnt, mask=last)
        if next_digit is not None:
            for c in range(K):
                nh = lax.bitwise_or(
                    lax.shift_left(lax.shift_right_logical(poss[c], 12), 11),
                    next_digit(ps[c]),
                )
                plsc.addupdate_scatter(gb, [nh], ones)

    def body(i, carry):
        nxt = load_dp(i + 1)
        process(carry)
        return nxt

    last_carry = lax.fori_loop(0, CHUNK_VREGS - 1, body, load_dp(0))
    process(last_carry)


def _d2(p):
    return lax.bitwise_and(p, jnp.int32(0x7FF))


def _d3(p):
    return lax.bitwise_and(p, jnp.int32(0x3FF))


def _body(x_hbm, out_hbm, xb, b, gb, in_sem, out_sem, *ga):
    ga = list(ga)
    cid = lax.axis_index("c")
    sid = lax.axis_index("s")
    wid = sid * 2 + cid

    zeros = jnp.zeros((16,), jnp.int32)

    @plsc.parallel_loop(0, K * NBINS // 16, unroll=8)
    def _zb_loop(i):
        gb[pl.ds(i * 16, 16)] = zeros

    pltpu.async_copy(x_hbm.at[wid * 4], xb, in_sem)

    def do_row(rr, _):
        row = wid * 4 + rr

        @plsc.parallel_loop(0, NBINS // 16, unroll=4)
        def _za_loop(i):
            sl = pl.ds(i * 16, 16)
            for c in range(K):
                ga[c][sl] = zeros

        pltpu.make_async_copy(x_hbm.at[row], xb, in_sem).wait()

        def src1(c, i):
            key = _keys(xb[pl.ds(c * CHUNK + i * 16, 16)])
            d = lax.bitwise_and(key, jnp.int32(0x7FF))
            p = lax.shift_right_logical(key, 11)
            if c < K // 2:
                p = lax.bitwise_or(p, jnp.int32(SIGN))
            return d, p

        def src2(c, i):
            w = b[pl.ds(c * CHUNK + i * 16, 16)]
            d = lax.bitwise_and(w, jnp.int32(0x7FF))
            p = lax.shift_right_logical(w, 11)
            return d, p

        def src3(c, i):
            w = plsc.bitcast(xb[pl.ds(c * CHUNK + i * 16, 16)], jnp.int32)
            d = lax.bitwise_and(w, jnp.int32(0x3FF))
            return d, lax.shift_right_logical(w, 20)

        ones = jnp.full((16,), 1, jnp.int32)

        @plsc.parallel_loop(0, CHUNK_VREGS, unroll=4)
        def _h1_loop(i):
            ds = [src1(c, i)[0] for c in range(K)]
            for c in range(K):
                plsc.addupdate_scatter(ga[c], [ds[c]], ones)
        _scan_ga_inplace(ga)

        @pl.when(rr > 0)
        def _():
            pltpu.make_async_copy(b, out_hbm.at[row], out_sem).wait()

        _perm_sweep(src1, b, ga, gb, _d2, cast_f32=False)

        _transfer_scan(gb, ga)
        _perm_sweep(src2, xb, ga, gb, _d3, cast_f32=True)

        _transfer_scan(gb, ga)
        _perm_sweep(src3, b, ga, None, None, cast_f32=False)

        pltpu.async_copy(b, out_hbm.at[row], out_sem)

        @pl.when(rr < ROWS // 32 - 1)
        def _():
            pltpu.async_copy(x_hbm.at[row + 1], xb, in_sem)

        return 0

    lax.fori_loop(0, ROWS // 32, do_row, 0)
    pltpu.make_async_copy(b, out_hbm.at[wid * 4 + ROWS // 32 - 1], out_sem).wait()


@jax.jit
def _feature_select(x):
    mesh = plsc.VectorSubcoreMesh(core_axis_name="c", subcore_axis_name="s")
    run = functools.partial(
        pl.kernel,
        out_type=jax.ShapeDtypeStruct((ROWS, N), jnp.int32),
        mesh=mesh,
        scratch_types=[
            pltpu.VMEM((N,), jnp.float32),
            pltpu.VMEM((N,), jnp.int32),
            pltpu.VMEM((K * NBINS,), jnp.int32),
            pltpu.SemaphoreType.DMA,
            pltpu.SemaphoreType.DMA,
        ]
        + [pltpu.VMEM((NBINS,), jnp.int32) for _ in range(K)],
        compiler_params=pltpu.CompilerParams(needs_layout_passes=False),
    )(_body)
    return run(x)


def kernel(x):
    return _feature_select(x)
